# baseline (speedup 1.0000x reference)
import functools

import jax
import jax.numpy as jnp
from jax import lax
from jax.experimental import pallas as pl
from jax.experimental.pallas import tpu as pltpu
from jax.experimental.pallas import tpu_sc as plsc

N = 10000
NPAD = 10240
E = 320000
D1 = 144
D2 = 16
NC = 2
NS = 16
NW = NC * NS
EPW = E // NW
C = 50
NCH = EPW // C
IB = 20
NIB = NCH // IB
RPS = NPAD // NS


def _f32(*shape):
    return jax.ShapeDtypeStruct(shape, jnp.float32)


def _sc_segsum(p_pad, e4, zeros, width, nbuf, sync_scat):
    mesh = plsc.VectorSubcoreMesh(core_axis_name="c", subcore_axis_name="s")
    n_ssem = 0 if sync_scat else nbuf

    @functools.partial(
        pl.kernel,
        out_type=_f32(2 * NPAD, width),
        mesh=mesh,
        scratch_types=(
            [pltpu.VMEM((IB, C), jnp.int32)] * 2
            + [pltpu.VMEM((C, width), jnp.float32)] * nbuf
            + [pltpu.VMEM_SHARED((NPAD, width), jnp.float32)]
            + [pltpu.SemaphoreType.DMA] * (nbuf + n_ssem)
        ),
        compiler_params=pltpu.CompilerParams(use_tc_tiling_on_sc=False),
    )
    def k(p_hbm, e_hbm, z_hbm, out_hbm, src_v, dst_v, *rest):
        bufs = rest[:nbuf]
        acc_sh = rest[nbuf]
        gsems = rest[nbuf + 1:2 * nbuf + 1]
        ssems = rest[2 * nbuf + 1:]
        cid = lax.axis_index("c")
        sid = lax.axis_index("s")
        wid = sid * NC + cid
        pltpu.sync_copy(z_hbm.at[pl.ds(sid * RPS, RPS)],
                        acc_sh.at[pl.ds(sid * RPS, RPS)])
        plsc.subcore_barrier()

        def gather(j, t):
            pltpu.async_copy(p_hbm.at[src_v.at[j]], bufs[t], gsems[t])

        def gdrain(t):
            pltpu.make_async_copy(p_hbm.at[src_v.at[0]], bufs[t],
                                  gsems[t]).wait()

        def sdrain(t):
            pltpu.make_async_copy(bufs[t], acc_sh.at[dst_v.at[0]],
                                  ssems[t]).wait()

        def block(b, carry):
            pltpu.sync_copy(e_hbm.at[0, wid, pl.ds(b * IB, IB)], src_v)
            pltpu.sync_copy(e_hbm.at[1, wid, pl.ds(b * IB, IB)], dst_v)
            for t in range(nbuf):
                gather(t, t)

            def body(i, c2):
                j = nbuf * i
                for t in range(nbuf):
                    gdrain(t)
                    if sync_scat:
                        pltpu.sync_copy(bufs[t], acc_sh.at[dst_v.at[j + t]],
                                        add=True)

                        def _nxt(t=t, j=j):
                            gather(j + t + nbuf, t)
                    else:
                        pltpu.async_copy(bufs[t], acc_sh.at[dst_v.at[j + t]],
                                         ssems[t], add=True)

                        def _nxt(t=t, j=j):
                            sdrain(t)
                            gather(j + t + nbuf, t)

                    pl.when(j + t + nbuf < IB)(_nxt)
                return c2

            lax.fori_loop(0, IB // nbuf, body, 0)
            if not sync_scat:
                for t in range(nbuf):
                    sdrain(t)
            return carry

        lax.fori_loop(0, NIB, block, 0)
        plsc.subcore_barrier()
        pltpu.sync_copy(acc_sh.at[pl.ds(sid * RPS, RPS)],
                        out_hbm.at[pl.ds(cid * NPAD + sid * RPS, RPS)])

    return k(p_pad, e4, zeros)


def _softplus(v):
    return jnp.maximum(v, 0.0) + jnp.log1p(jnp.exp(-jnp.abs(v)))


def _tc_root(h, wr, b):
    def body(h_ref, wr_ref, b_ref, o_ref):
        o_ref[...] = jnp.dot(h_ref[...], wr_ref[...],
                             preferred_element_type=jnp.float32) + b_ref[...]

    return pl.pallas_call(body, out_shape=_f32(N, D2))(h, wr, b)


def _tc_mid(parts, root1, w1p):
    def body(parts_ref, r_ref, w_ref, o_ref):
        s = parts_ref[0:N, :] + parts_ref[NPAD:NPAD + N, :]
        den = jnp.maximum(s[:, 128:129], 1.0)
        mean1 = s[:, 0:128] / den
        h1 = _softplus(jnp.dot(mean1, w_ref[...],
                               preferred_element_type=jnp.float32)
                       + r_ref[...])
        lane = lax.broadcasted_iota(jnp.int32, (N, D2), 1)
        o_ref[...] = jnp.where(lane == D2 - 1, 1.0, h1)

    return pl.pallas_call(
        body, out_shape=_f32(N, D2),
    )(parts, root1, w1p)


def _tc_head(parts, root2, w2p, fc1p, fc1bp, fc2p, fc2bp):
    def body(parts_ref, r_ref, w_ref, w3_ref, b3_ref, w4_ref, b4_ref, o_ref):
        s = parts_ref[0:N, :] + parts_ref[NPAD:NPAD + N, :]
        den = jnp.maximum(s[:, D2 - 1:D2], 1.0)
        mean2 = s / den
        h2 = _softplus(jnp.dot(mean2, w_ref[...],
                               preferred_element_type=jnp.float32)
                       + r_ref[...])
        h3 = _softplus(jnp.dot(h2, w3_ref[...],
                               preferred_element_type=jnp.float32)
                       + b3_ref[...])
        oo = jnp.dot(h3, w4_ref[...],
                     preferred_element_type=jnp.float32) + b4_ref[...]
        col = oo[:, 0:1]
        o_ref[...] = col - jnp.min(col)

    return pl.pallas_call(
        body, out_shape=_f32(N, 1),
    )(parts, root2, w2p, fc1p, fc1bp, fc2p, fc2bp)


def _pad2(a, rows, cols):
    return jnp.pad(a, ((0, rows - a.shape[0]), (0, cols - a.shape[1])))


def kernel(x, edge_index, W_rel1, b_rel1, W_root1, W_rel2, b_rel2, W_root2,
           fc1_W, fc1_b, fc2_W, fc2_b):
    e4 = edge_index.reshape(2, NW, NCH, C)
    zeros1 = jnp.zeros((NPAD, D1), jnp.float32)
    zeros2 = jnp.zeros((NPAD, D2), jnp.float32)

    w1p = _pad2(W_rel1, 128, D2)
    wr1p = _pad2(W_root1, 128, D2)
    b1p = _pad2(b_rel1[None, :], 1, D2)
    w2p = _pad2(W_rel2, D2, D2)
    wr2p = _pad2(W_root2, D2, D2)
    b2p = _pad2(b_rel2[None, :], 1, D2)
    fc1p = _pad2(fc1_W, D2, 128)
    fc1bp = _pad2(fc1_b[None, :], 1, 128)
    fc2p = _pad2(fc2_W, 128, 8)
    fc2bp = _pad2(fc2_b[None, :], 1, 8)

    xp = jnp.concatenate(
        [x, jnp.ones((N, 1), jnp.float32), jnp.zeros((N, D1 - 129), jnp.float32)],
        axis=1)
    parts1 = _sc_segsum(xp, e4, zeros1, D1, 4, False)
    root1 = _tc_root(x, wr1p, b1p)
    h1p = _tc_mid(parts1, root1, w1p)
    parts2 = _sc_segsum(h1p, e4, zeros2, D2, 4, False)
    root2 = _tc_root(h1p, wr2p, b2p)
    return _tc_head(parts2, root2, w2p, fc1p, fc1bp, fc2p, fc2bp)


# speedup vs baseline: 1.1064x; 1.1064x over previous
import functools

import jax
import jax.numpy as jnp
from jax import lax
from jax.experimental import pallas as pl
from jax.experimental.pallas import tpu as pltpu
from jax.experimental.pallas import tpu_sc as plsc

N = 10000
NPAD = 10240
E = 320000
D1 = 144
D2 = 16
NC = 2
NS = 16
NW = NC * NS
EPW = E // NW
C = 100
NCH = EPW // C
IB = 20
NIB = NCH // IB
RPS = NPAD // NS


def _f32(*shape):
    return jax.ShapeDtypeStruct(shape, jnp.float32)


def _sc_segsum(p_pad, e4, zeros, width, nbuf, sync_scat):
    mesh = plsc.VectorSubcoreMesh(core_axis_name="c", subcore_axis_name="s")
    n_ssem = 0 if sync_scat else nbuf

    @functools.partial(
        pl.kernel,
        out_type=_f32(2 * NPAD, width),
        mesh=mesh,
        scratch_types=(
            [pltpu.VMEM((IB, C), jnp.int32)] * 2
            + [pltpu.VMEM((C, width), jnp.float32)] * nbuf
            + [pltpu.VMEM_SHARED((NPAD, width), jnp.float32)]
            + [pltpu.SemaphoreType.DMA] * (nbuf + n_ssem)
        ),
        compiler_params=pltpu.CompilerParams(use_tc_tiling_on_sc=False),
    )
    def k(p_hbm, e_hbm, z_hbm, out_hbm, src_v, dst_v, *rest):
        bufs = rest[:nbuf]
        acc_sh = rest[nbuf]
        gsems = rest[nbuf + 1:2 * nbuf + 1]
        ssems = rest[2 * nbuf + 1:]
        cid = lax.axis_index("c")
        sid = lax.axis_index("s")
        wid = sid * NC + cid
        pltpu.sync_copy(z_hbm.at[pl.ds(sid * RPS, RPS)],
                        acc_sh.at[pl.ds(sid * RPS, RPS)])
        plsc.subcore_barrier()

        def gather(j, t):
            pltpu.async_copy(p_hbm.at[src_v.at[j]], bufs[t], gsems[t])

        def gdrain(t):
            pltpu.make_async_copy(p_hbm.at[src_v.at[0]], bufs[t],
                                  gsems[t]).wait()

        def sdrain(t):
            pltpu.make_async_copy(bufs[t], acc_sh.at[dst_v.at[0]],
                                  ssems[t]).wait()

        def block(b, carry):
            pltpu.sync_copy(e_hbm.at[0, wid, pl.ds(b * IB, IB)], src_v)
            pltpu.sync_copy(e_hbm.at[1, wid, pl.ds(b * IB, IB)], dst_v)
            for t in range(nbuf):
                gather(t, t)

            def body(i, c2):
                j = nbuf * i
                for t in range(nbuf):
                    gdrain(t)
                    if sync_scat:
                        pltpu.sync_copy(bufs[t], acc_sh.at[dst_v.at[j + t]],
                                        add=True)

                        def _nxt(t=t, j=j):
                            gather(j + t + nbuf, t)
                    else:
                        pltpu.async_copy(bufs[t], acc_sh.at[dst_v.at[j + t]],
                                         ssems[t], add=True)

                        def _nxt(t=t, j=j):
                            sdrain(t)
                            gather(j + t + nbuf, t)

                    pl.when(j + t + nbuf < IB)(_nxt)
                return c2

            lax.fori_loop(0, IB // nbuf, body, 0)
            if not sync_scat:
                for t in range(nbuf):
                    sdrain(t)
            return carry

        lax.fori_loop(0, NIB, block, 0)
        plsc.subcore_barrier()
        pltpu.sync_copy(acc_sh.at[pl.ds(sid * RPS, RPS)],
                        out_hbm.at[pl.ds(cid * NPAD + sid * RPS, RPS)])

    return k(p_pad, e4, zeros)


def _softplus(v):
    return jnp.maximum(v, 0.0) + jnp.log1p(jnp.exp(-jnp.abs(v)))


def _tc_root(h, wr, b):
    def body(h_ref, wr_ref, b_ref, o_ref):
        o_ref[...] = jnp.dot(h_ref[...], wr_ref[...],
                             preferred_element_type=jnp.float32) + b_ref[...]

    return pl.pallas_call(body, out_shape=_f32(N, D2))(h, wr, b)


def _tc_mid(parts, root1, w1p):
    def body(parts_ref, r_ref, w_ref, o_ref):
        s = parts_ref[0:N, :] + parts_ref[NPAD:NPAD + N, :]
        den = jnp.maximum(s[:, 128:129], 1.0)
        mean1 = s[:, 0:128] / den
        h1 = _softplus(jnp.dot(mean1, w_ref[...],
                               preferred_element_type=jnp.float32)
                       + r_ref[...])
        lane = lax.broadcasted_iota(jnp.int32, (N, D2), 1)
        o_ref[...] = jnp.where(lane == D2 - 1, 1.0, h1)

    return pl.pallas_call(
        body, out_shape=_f32(N, D2),
    )(parts, root1, w1p)


def _tc_head(parts, root2, w2p, fc1p, fc1bp, fc2p, fc2bp):
    def body(parts_ref, r_ref, w_ref, w3_ref, b3_ref, w4_ref, b4_ref, o_ref):
        s = parts_ref[0:N, :] + parts_ref[NPAD:NPAD + N, :]
        den = jnp.maximum(s[:, D2 - 1:D2], 1.0)
        mean2 = s / den
        h2 = _softplus(jnp.dot(mean2, w_ref[...],
                               preferred_element_type=jnp.float32)
                       + r_ref[...])
        h3 = _softplus(jnp.dot(h2, w3_ref[...],
                               preferred_element_type=jnp.float32)
                       + b3_ref[...])
        oo = jnp.dot(h3, w4_ref[...],
                     preferred_element_type=jnp.float32) + b4_ref[...]
        col = oo[:, 0:1]
        o_ref[...] = col - jnp.min(col)

    return pl.pallas_call(
        body, out_shape=_f32(N, 1),
    )(parts, root2, w2p, fc1p, fc1bp, fc2p, fc2bp)


def _pad2(a, rows, cols):
    return jnp.pad(a, ((0, rows - a.shape[0]), (0, cols - a.shape[1])))


def kernel(x, edge_index, W_rel1, b_rel1, W_root1, W_rel2, b_rel2, W_root2,
           fc1_W, fc1_b, fc2_W, fc2_b):
    e4 = edge_index.reshape(2, NW, NCH, C)
    zeros1 = jnp.zeros((NPAD, D1), jnp.float32)
    zeros2 = jnp.zeros((NPAD, D2), jnp.float32)

    w1p = _pad2(W_rel1, 128, D2)
    wr1p = _pad2(W_root1, 128, D2)
    b1p = _pad2(b_rel1[None, :], 1, D2)
    w2p = _pad2(W_rel2, D2, D2)
    wr2p = _pad2(W_root2, D2, D2)
    b2p = _pad2(b_rel2[None, :], 1, D2)
    fc1p = _pad2(fc1_W, D2, 128)
    fc1bp = _pad2(fc1_b[None, :], 1, 128)
    fc2p = _pad2(fc2_W, 128, 8)
    fc2bp = _pad2(fc2_b[None, :], 1, 8)

    xp = jnp.concatenate(
        [x, jnp.ones((N, 1), jnp.float32), jnp.zeros((N, D1 - 129), jnp.float32)],
        axis=1)
    parts1 = _sc_segsum(xp, e4, zeros1, D1, 2, True)
    root1 = _tc_root(x, wr1p, b1p)
    h1p = _tc_mid(parts1, root1, w1p)
    parts2 = _sc_segsum(h1p, e4, zeros2, D2, 4, False)
    root2 = _tc_root(h1p, wr2p, b2p)
    return _tc_head(parts2, root2, w2p, fc1p, fc1bp, fc2p, fc2bp)


# speedup vs baseline: 1.1117x; 1.0048x over previous
import functools

import jax
import jax.numpy as jnp
from jax import lax
from jax.experimental import pallas as pl
from jax.experimental.pallas import tpu as pltpu
from jax.experimental.pallas import tpu_sc as plsc

N = 10000
NPAD = 10240
E = 320000
D1 = 144
D2 = 16
NC = 2
NS = 16
NW = NC * NS
EPW = E // NW
C = 100
NCH = EPW // C
IB = 20
NIB = NCH // IB
RPS = NPAD // NS


def _f32(*shape):
    return jax.ShapeDtypeStruct(shape, jnp.float32)


def _sc_segsum(p_pad, e4, zeros, width, nbuf, sync_scat):
    mesh = plsc.VectorSubcoreMesh(core_axis_name="c", subcore_axis_name="s")
    n_ssem = 0 if sync_scat else nbuf

    @functools.partial(
        pl.kernel,
        out_type=_f32(2 * NPAD, width),
        mesh=mesh,
        scratch_types=(
            [pltpu.VMEM((IB, C), jnp.int32)] * 2
            + [pltpu.VMEM((C, width), jnp.float32)] * nbuf
            + [pltpu.VMEM_SHARED((NPAD, width), jnp.float32)]
            + [pltpu.SemaphoreType.DMA] * (nbuf + n_ssem)
        ),
        compiler_params=pltpu.CompilerParams(use_tc_tiling_on_sc=False),
    )
    def k(p_hbm, e_hbm, z_hbm, out_hbm, src_v, dst_v, *rest):
        bufs = rest[:nbuf]
        acc_sh = rest[nbuf]
        gsems = rest[nbuf + 1:2 * nbuf + 1]
        ssems = rest[2 * nbuf + 1:]
        cid = lax.axis_index("c")
        sid = lax.axis_index("s")
        wid = sid * NC + cid
        pltpu.sync_copy(z_hbm.at[pl.ds(sid * RPS, RPS)],
                        acc_sh.at[pl.ds(sid * RPS, RPS)])
        plsc.subcore_barrier()

        def gather(j, t):
            pltpu.async_copy(p_hbm.at[src_v.at[j]], bufs[t], gsems[t])

        def gdrain(t):
            pltpu.make_async_copy(p_hbm.at[src_v.at[0]], bufs[t],
                                  gsems[t]).wait()

        def sdrain(t):
            pltpu.make_async_copy(bufs[t], acc_sh.at[dst_v.at[0]],
                                  ssems[t]).wait()

        def block(b, carry):
            pltpu.sync_copy(e_hbm.at[0, wid, pl.ds(b * IB, IB)], src_v)
            pltpu.sync_copy(e_hbm.at[1, wid, pl.ds(b * IB, IB)], dst_v)
            for t in range(nbuf):
                gather(t, t)

            def body(i, c2):
                j = nbuf * i
                for t in range(nbuf):
                    gdrain(t)
                    if sync_scat:
                        pltpu.sync_copy(bufs[t], acc_sh.at[dst_v.at[j + t]],
                                        add=True)

                        def _nxt(t=t, j=j):
                            gather(j + t + nbuf, t)
                    else:
                        pltpu.async_copy(bufs[t], acc_sh.at[dst_v.at[j + t]],
                                         ssems[t], add=True)

                        def _nxt(t=t, j=j):
                            sdrain(t)
                            gather(j + t + nbuf, t)

                    pl.when(j + t + nbuf < IB)(_nxt)
                return c2

            lax.fori_loop(0, IB // nbuf, body, 0)
            if not sync_scat:
                for t in range(nbuf):
                    sdrain(t)
            return carry

        lax.fori_loop(0, NIB, block, 0)
        plsc.subcore_barrier()
        pltpu.sync_copy(acc_sh.at[pl.ds(sid * RPS, RPS)],
                        out_hbm.at[pl.ds(cid * NPAD + sid * RPS, RPS)])

    return k(p_pad, e4, zeros)


def _softplus(v):
    return jnp.maximum(v, 0.0) + jnp.log1p(jnp.exp(-jnp.abs(v)))


def _tc_mid(parts, x, w1p, wr1p, b1p):
    def body(parts_ref, x_ref, w_ref, wr_ref, b_ref, o_ref):
        s = parts_ref[0:N, :] + parts_ref[NPAD:NPAD + N, :]
        den = jnp.maximum(s[:, 128:129], 1.0)
        mean1 = s[:, 0:128] / den
        h1 = _softplus(jnp.dot(mean1, w_ref[...],
                               preferred_element_type=jnp.float32)
                       + b_ref[...]
                       + jnp.dot(x_ref[...], wr_ref[...],
                                 preferred_element_type=jnp.float32))
        lane = lax.broadcasted_iota(jnp.int32, (N, D2), 1)
        o_ref[...] = jnp.where(lane == D2 - 1, 1.0, h1)

    return pl.pallas_call(
        body, out_shape=_f32(N, D2),
    )(parts, x, w1p, wr1p, b1p)


def _tc_head(parts, h1p, w2p, wr2p, b2p, fc1p, fc1bp, fc2p, fc2bp):
    def body(parts_ref, h1_ref, w_ref, wr_ref, b_ref, w3_ref, b3_ref,
             w4_ref, b4_ref, o_ref):
        s = parts_ref[0:N, :] + parts_ref[NPAD:NPAD + N, :]
        den = jnp.maximum(s[:, D2 - 1:D2], 1.0)
        mean2 = s / den
        h2 = _softplus(jnp.dot(mean2, w_ref[...],
                               preferred_element_type=jnp.float32)
                       + b_ref[...]
                       + jnp.dot(h1_ref[...], wr_ref[...],
                                 preferred_element_type=jnp.float32))
        h3 = _softplus(jnp.dot(h2, w3_ref[...],
                               preferred_element_type=jnp.float32)
                       + b3_ref[...])
        oo = jnp.dot(h3, w4_ref[...],
                     preferred_element_type=jnp.float32) + b4_ref[...]
        col = oo[:, 0:1]
        o_ref[...] = col - jnp.min(col)

    return pl.pallas_call(
        body, out_shape=_f32(N, 1),
    )(parts, h1p, w2p, wr2p, b2p, fc1p, fc1bp, fc2p, fc2bp)


def _pad2(a, rows, cols):
    return jnp.pad(a, ((0, rows - a.shape[0]), (0, cols - a.shape[1])))


def kernel(x, edge_index, W_rel1, b_rel1, W_root1, W_rel2, b_rel2, W_root2,
           fc1_W, fc1_b, fc2_W, fc2_b):
    e4 = edge_index.reshape(2, NW, NCH, C)
    zeros1 = jnp.zeros((NPAD, D1), jnp.float32)
    zeros2 = jnp.zeros((NPAD, D2), jnp.float32)

    w1p = _pad2(W_rel1, 128, D2)
    wr1p = _pad2(W_root1, 128, D2)
    b1p = _pad2(b_rel1[None, :], 1, D2)
    w2p = _pad2(W_rel2, D2, D2)
    wr2p = _pad2(W_root2, D2, D2)
    b2p = _pad2(b_rel2[None, :], 1, D2)
    fc1p = _pad2(fc1_W, D2, 128)
    fc1bp = _pad2(fc1_b[None, :], 1, 128)
    fc2p = _pad2(fc2_W, 128, 8)
    fc2bp = _pad2(fc2_b[None, :], 1, 8)

    xp = jnp.concatenate(
        [x, jnp.ones((N, 1), jnp.float32), jnp.zeros((N, D1 - 129), jnp.float32)],
        axis=1)
    parts1 = _sc_segsum(xp, e4, zeros1, D1, 2, True)
    h1p = _tc_mid(parts1, x, w1p, wr1p, b1p)
    parts2 = _sc_segsum(h1p, e4, zeros2, D2, 4, False)
    return _tc_head(parts2, h1p, w2p, wr2p, b2p, fc1p, fc1bp, fc2p, fc2bp)
